# baseline (speedup 1.0000x reference)
import functools

import jax
import jax.numpy as jnp
import numpy as np
from jax.experimental import pallas as pl
from jax.experimental.pallas import tpu as pltpu

_ZERO_PATCH_CACHE = {}

_NCHUNK = 4


def _zero_patch(b, d):
    if (b, d) not in _ZERO_PATCH_CACHE:
        with jax.ensure_compile_time_eval():
            bern = jax.random.bernoulli(jax.random.key(1), 0.5, (b, d))
            arr = 2 * bern.astype(jnp.int8) - 1
        _ZERO_PATCH_CACHE[(b, d)] = np.asarray(arr)
    return _ZERO_PATCH_CACHE[(b, d)]


def _encoder_kernel(x_hbm, iv_ref, id_ref, lvl_ref, ones_hbm, out_hbm,
                    xv, ov, outv, w_ref, sx, so, sw):
    q = iv_ref.shape[1]
    f = id_ref.shape[0]
    n = xv.shape[0]
    ch = xv.shape[1]

    for c in range(n):
        pltpu.make_async_copy(x_hbm.at[pl.ds(c * ch, ch), :], xv.at[c],
                              sx.at[c]).start()
        pltpu.make_async_copy(ones_hbm.at[pl.ds(c * ch, ch), :], ov.at[c],
                              so.at[c]).start()

    idb = id_ref[...].astype(jnp.bfloat16)
    for l in range(q):
        diff = (lvl_ref[l + 1:l + 2, :] -
                lvl_ref[l:l + 1, :]).astype(jnp.bfloat16)
        w_ref[l * f:(l + 1) * f, :] = idb * diff
    base = lvl_ref[0:1, :] * jnp.sum(id_ref[...], axis=0, keepdims=True)

    for c in range(n):
        pltpu.make_async_copy(x_hbm.at[pl.ds(c * ch, ch), :], xv.at[c],
                              sx.at[c]).wait()
        xc = jnp.clip(xv[c], -1.0, 1.0)
        cmp = jnp.concatenate(
            [(xc > iv_ref[0, l]).astype(jnp.bfloat16) for l in range(q)],
            axis=1)
        enc = jnp.clip(
            jnp.dot(cmp, w_ref[...], preferred_element_type=jnp.float32) +
            base, -1.0, 1.0)
        pltpu.make_async_copy(ones_hbm.at[pl.ds(c * ch, ch), :], ov.at[c],
                              so.at[c]).wait()
        outv[c] = jnp.where(enc == 0.0, ov[c].astype(jnp.float32), enc)
        pltpu.make_async_copy(outv.at[c], out_hbm.at[pl.ds(c * ch, ch), :],
                              sw.at[c]).start()

    for c in range(n):
        pltpu.make_async_copy(outv.at[c], out_hbm.at[pl.ds(c * ch, ch), :],
                              sw.at[c]).wait()


def kernel(x, intervals, id_hvs, lvl_hvs, interpret=False):
    b, dim_in = x.shape
    d = id_hvs.shape[1]
    q = intervals.shape[0]
    levels = lvl_hvs.shape[0]
    ones = jnp.asarray(_zero_patch(b, d))
    iv2d = intervals.reshape(1, q)

    n = _NCHUNK if b % _NCHUNK == 0 else 1
    ch = b // n
    return pl.pallas_call(
        _encoder_kernel,
        grid=(1,),
        in_specs=[
            pl.BlockSpec(memory_space=pltpu.MemorySpace.HBM),
            pl.BlockSpec((1, q), lambda i: (0, 0)),
            pl.BlockSpec((dim_in, d), lambda i: (0, 0)),
            pl.BlockSpec((levels, d), lambda i: (0, 0)),
            pl.BlockSpec(memory_space=pltpu.MemorySpace.HBM),
        ],
        out_specs=pl.BlockSpec(memory_space=pltpu.MemorySpace.HBM),
        out_shape=jax.ShapeDtypeStruct((b, d), jnp.float32),
        scratch_shapes=[
            pltpu.VMEM((n, ch, dim_in), jnp.float32),
            pltpu.VMEM((n, ch, d), jnp.int8),
            pltpu.VMEM((n, ch, d), jnp.float32),
            pltpu.VMEM((q * dim_in, d), jnp.bfloat16),
            pltpu.SemaphoreType.DMA((n,)),
            pltpu.SemaphoreType.DMA((n,)),
            pltpu.SemaphoreType.DMA((n,)),
        ],
        interpret=interpret,
    )(x, iv2d, id_hvs, lvl_hvs, ones)


# speedup vs baseline: 1.0602x; 1.0602x over previous
import functools

import jax
import jax.numpy as jnp
import numpy as np
from jax.experimental import pallas as pl
from jax.experimental.pallas import tpu as pltpu

_ZERO_PATCH_CACHE = {}

_NCHUNK = 2


def _zero_patch(b, d):
    if (b, d) not in _ZERO_PATCH_CACHE:
        with jax.ensure_compile_time_eval():
            bern = jax.random.bernoulli(jax.random.key(1), 0.5, (b, d))
            arr = 2 * bern.astype(jnp.int8) - 1
        _ZERO_PATCH_CACHE[(b, d)] = np.asarray(arr)
    return _ZERO_PATCH_CACHE[(b, d)]


def _encoder_kernel(x_hbm, iv_ref, id_ref, lvl_ref, ones_hbm, out_hbm,
                    xv, ov, outv, w_ref, sx, so, sw):
    q = iv_ref.shape[1]
    f = id_ref.shape[0]
    n = xv.shape[0]
    ch = xv.shape[1]

    for c in range(n):
        pltpu.make_async_copy(x_hbm.at[pl.ds(c * ch, ch), :], xv.at[c],
                              sx.at[c]).start()
        pltpu.make_async_copy(ones_hbm.at[pl.ds(c * ch, ch), :], ov.at[c],
                              so.at[c]).start()

    idb = id_ref[...].astype(jnp.bfloat16)
    for l in range(q):
        diff = (lvl_ref[l + 1:l + 2, :] -
                lvl_ref[l:l + 1, :]).astype(jnp.bfloat16)
        w_ref[l * f:(l + 1) * f, :] = idb * diff
    base = lvl_ref[0:1, :] * jnp.sum(id_ref[...], axis=0, keepdims=True)

    for c in range(n):
        pltpu.make_async_copy(x_hbm.at[pl.ds(c * ch, ch), :], xv.at[c],
                              sx.at[c]).wait()
        xc = jnp.clip(xv[c], -1.0, 1.0)
        cmp = jnp.concatenate(
            [(xc > iv_ref[0, l]).astype(jnp.bfloat16) for l in range(q)],
            axis=1)
        enc = jnp.clip(
            jnp.dot(cmp, w_ref[...], preferred_element_type=jnp.float32) +
            base, -1.0, 1.0)
        pltpu.make_async_copy(ones_hbm.at[pl.ds(c * ch, ch), :], ov.at[c],
                              so.at[c]).wait()
        outv[c] = jnp.where(enc == 0.0, ov[c].astype(jnp.float32), enc)
        pltpu.make_async_copy(outv.at[c], out_hbm.at[pl.ds(c * ch, ch), :],
                              sw.at[c]).start()

    for c in range(n):
        pltpu.make_async_copy(outv.at[c], out_hbm.at[pl.ds(c * ch, ch), :],
                              sw.at[c]).wait()


def kernel(x, intervals, id_hvs, lvl_hvs, interpret=False):
    b, dim_in = x.shape
    d = id_hvs.shape[1]
    q = intervals.shape[0]
    levels = lvl_hvs.shape[0]
    ones = jnp.asarray(_zero_patch(b, d))
    iv2d = intervals.reshape(1, q)

    n = _NCHUNK if b % _NCHUNK == 0 else 1
    ch = b // n
    return pl.pallas_call(
        _encoder_kernel,
        grid=(1,),
        in_specs=[
            pl.BlockSpec(memory_space=pltpu.MemorySpace.HBM),
            pl.BlockSpec((1, q), lambda i: (0, 0)),
            pl.BlockSpec((dim_in, d), lambda i: (0, 0)),
            pl.BlockSpec((levels, d), lambda i: (0, 0)),
            pl.BlockSpec(memory_space=pltpu.MemorySpace.HBM),
        ],
        out_specs=pl.BlockSpec(memory_space=pltpu.MemorySpace.HBM),
        out_shape=jax.ShapeDtypeStruct((b, d), jnp.float32),
        scratch_shapes=[
            pltpu.VMEM((n, ch, dim_in), jnp.float32),
            pltpu.VMEM((n, ch, d), jnp.int8),
            pltpu.VMEM((n, ch, d), jnp.float32),
            pltpu.VMEM((q * dim_in, d), jnp.bfloat16),
            pltpu.SemaphoreType.DMA((n,)),
            pltpu.SemaphoreType.DMA((n,)),
            pltpu.SemaphoreType.DMA((n,)),
        ],
        interpret=interpret,
    )(x, iv2d, id_hvs, lvl_hvs, ones)


# speedup vs baseline: 1.2438x; 1.1732x over previous
import functools

import jax
import jax.numpy as jnp
import numpy as np
from jax.experimental import pallas as pl
from jax.experimental.pallas import tpu as pltpu

_ZERO_PATCH_CACHE = {}


def _zero_patch(b, d):
    if (b, d) not in _ZERO_PATCH_CACHE:
        with jax.ensure_compile_time_eval():
            bern = jax.random.bernoulli(jax.random.key(1), 0.5, (b, d))
            arr = 2 * bern.astype(jnp.int8) - 1
        _ZERO_PATCH_CACHE[(b, d)] = np.asarray(arr)
    return _ZERO_PATCH_CACHE[(b, d)]


def _encoder_kernel(x_ref, iv_ref, id_ref, lvl_ref, ones_ref, out_ref,
                    w_ref):
    q = iv_ref.shape[1]
    f = id_ref.shape[0]

    @pl.when(pl.program_id(0) == 0)
    def _init():
        idb = id_ref[...].astype(jnp.bfloat16)
        for l in range(q):
            diff = (lvl_ref[l + 1:l + 2, :] -
                    lvl_ref[l:l + 1, :]).astype(jnp.bfloat16)
            w_ref[l * f:(l + 1) * f, :] = idb * diff

    xc = jnp.clip(x_ref[...], -1.0, 1.0)
    cmp = jnp.concatenate(
        [(xc > iv_ref[0, l]).astype(jnp.bfloat16) for l in range(q)], axis=1)
    base = lvl_ref[0:1, :] * jnp.sum(id_ref[...], axis=0, keepdims=True)
    enc = jnp.clip(
        jnp.dot(cmp, w_ref[...], preferred_element_type=jnp.float32) + base,
        -1.0, 1.0)
    out_ref[...] = jnp.where(enc == 0.0, ones_ref[...].astype(jnp.float32),
                             enc)


def kernel(x, intervals, id_hvs, lvl_hvs, interpret=False):
    b, dim_in = x.shape
    d = id_hvs.shape[1]
    q = intervals.shape[0]
    levels = lvl_hvs.shape[0]
    ones = jnp.asarray(_zero_patch(b, d))
    iv2d = intervals.reshape(1, q)

    blk = b
    grid = (b // blk,)
    return pl.pallas_call(
        _encoder_kernel,
        grid=grid,
        in_specs=[
            pl.BlockSpec((blk, dim_in), lambda i: (i, 0)),
            pl.BlockSpec((1, q), lambda i: (0, 0)),
            pl.BlockSpec((dim_in, d), lambda i: (0, 0)),
            pl.BlockSpec((levels, d), lambda i: (0, 0)),
            pl.BlockSpec((blk, d), lambda i: (i, 0)),
        ],
        out_specs=pl.BlockSpec((blk, d), lambda i: (i, 0)),
        out_shape=jax.ShapeDtypeStruct((b, d), jnp.float32),
        scratch_shapes=[
            pltpu.VMEM((q * dim_in, d), jnp.bfloat16),
        ],
        interpret=interpret,
    )(x, iv2d, id_hvs, lvl_hvs, ones)


# speedup vs baseline: 1.2470x; 1.0026x over previous
import jax
import jax.numpy as jnp
import numpy as np
from jax.experimental import pallas as pl
from jax.experimental.pallas import tpu as pltpu

_ZERO_PATCH_CACHE = {}


def _zero_patch(b, d):
    if (b, d) not in _ZERO_PATCH_CACHE:
        with jax.ensure_compile_time_eval():
            bern = jax.random.bernoulli(jax.random.key(1), 0.5, (b, d))
            arr = 2 * bern.astype(jnp.int8) - 1
        _ZERO_PATCH_CACHE[(b, d)] = np.asarray(arr)
    return _ZERO_PATCH_CACHE[(b, d)]


def _encoder_kernel(x_ref, iv_ref, id_ref, lvl_ref, ones_ref, out_ref,
                    w_ref):
    q = iv_ref.shape[1]
    f = id_ref.shape[0]

    @pl.when(pl.program_id(0) == 0)
    def _init():
        idb = id_ref[...].astype(jnp.bfloat16)
        for l in range(q):
            diff = (lvl_ref[l + 1:l + 2, :] -
                    lvl_ref[l:l + 1, :]).astype(jnp.bfloat16)
            w_ref[l * f:(l + 1) * f, :] = idb * diff

    xc = jnp.clip(x_ref[...], -1.0, 1.0)
    cmp = jnp.concatenate(
        [(xc > iv_ref[0, l]).astype(jnp.bfloat16) for l in range(q)], axis=1)
    base = lvl_ref[0:1, :] * jnp.sum(id_ref[...], axis=0, keepdims=True)
    enc = jnp.clip(
        jnp.dot(cmp, w_ref[...], preferred_element_type=jnp.float32) + base,
        -1.0, 1.0)
    out_ref[...] = jnp.where(enc == 0.0, ones_ref[...].astype(jnp.float32),
                             enc)


def kernel(x, intervals, id_hvs, lvl_hvs):
    b, dim_in = x.shape
    d = id_hvs.shape[1]
    q = intervals.shape[0]
    levels = lvl_hvs.shape[0]
    ones = jnp.asarray(_zero_patch(b, d))
    iv2d = intervals.reshape(1, q)

    blk = b
    grid = (b // blk,)
    return pl.pallas_call(
        _encoder_kernel,
        grid=grid,
        in_specs=[
            pl.BlockSpec((blk, dim_in), lambda i: (i, 0)),
            pl.BlockSpec((1, q), lambda i: (0, 0)),
            pl.BlockSpec((dim_in, d), lambda i: (0, 0)),
            pl.BlockSpec((levels, d), lambda i: (0, 0)),
            pl.BlockSpec((blk, d), lambda i: (i, 0)),
        ],
        out_specs=pl.BlockSpec((blk, d), lambda i: (i, 0)),
        out_shape=jax.ShapeDtypeStruct((b, d), jnp.float32),
        scratch_shapes=[
            pltpu.VMEM((q * dim_in, d), jnp.bfloat16),
        ],
    )(x, iv2d, id_hvs, lvl_hvs, ones)
